# Initial kernel scaffold; baseline (speedup 1.0000x reference)
import jax
import jax.numpy as jnp
from jax.experimental import pallas as pl


def kernel(x, w, k):
    raise NotImplementedError("write your pallas kernel here")


# speedup vs baseline: 4.8661x; 4.8661x over previous
import jax
import jax.numpy as jnp
from jax.experimental import pallas as pl
from jax.experimental.pallas import tpu as pltpu

_BN = 2048


def _mv_body(w_ref, x_ref, o_ref, sw_ref):
    i = pl.program_id(0)

    @pl.when(i == 0)
    def _():
        wv = w_ref[...]
        m = jnp.max(wv)
        e = jnp.exp(wv - m)
        sw_ref[...] = e / jnp.sum(e)
        o_ref[...] = jnp.zeros_like(o_ref)

    bn = x_ref.shape[1]
    swb = sw_ref[0:1, pl.ds(i * bn, bn)]
    o_ref[...] += jnp.sum(x_ref[...] * swb, axis=1, keepdims=True)


def kernel(x, w, k):
    del k
    t, n = x.shape
    bn = min(_BN, n)
    y = pl.pallas_call(
        _mv_body,
        grid=(n // bn,),
        in_specs=[
            pl.BlockSpec((1, n), lambda i: (0, 0)),
            pl.BlockSpec((t, bn), lambda i: (0, i)),
        ],
        out_specs=pl.BlockSpec((t, 1), lambda i: (0, 0)),
        out_shape=jax.ShapeDtypeStruct((t, 1), jnp.float32),
        scratch_shapes=[pltpu.VMEM((1, n), jnp.float32)],
    )(w.reshape(1, n), x)
    return y.reshape(t)


# speedup vs baseline: 4.9191x; 1.0109x over previous
import jax
import jax.numpy as jnp
from jax.experimental import pallas as pl
from jax.experimental.pallas import tpu as pltpu

_BT = 128


def _mv_body(w_ref, x_ref, o_ref, sw_ref):
    i = pl.program_id(0)

    @pl.when(i == 0)
    def _():
        wv = w_ref[...]
        m = jnp.max(wv)
        e = jnp.exp(wv - m)
        sw_ref[...] = e / jnp.sum(e)

    o_ref[...] = jnp.sum(x_ref[...] * sw_ref[...], axis=1, keepdims=True)


def kernel(x, w, k):
    del k
    t, n = x.shape
    bt = min(_BT, t)
    y = pl.pallas_call(
        _mv_body,
        grid=(t // bt,),
        in_specs=[
            pl.BlockSpec((1, n), lambda i: (0, 0)),
            pl.BlockSpec((bt, n), lambda i: (i, 0)),
        ],
        out_specs=pl.BlockSpec((bt, 1), lambda i: (i, 0)),
        out_shape=jax.ShapeDtypeStruct((t, 1), jnp.float32),
        scratch_shapes=[pltpu.VMEM((1, n), jnp.float32)],
    )(w.reshape(1, n), x)
    return y.reshape(t)


# speedup vs baseline: 4.9620x; 1.0087x over previous
import jax
import jax.numpy as jnp
from jax.experimental import pallas as pl
from jax.experimental.pallas import tpu as pltpu

_BT = 64


def _mv_body(w_ref, x_ref, o_ref, sw_ref):
    i = pl.program_id(0)

    @pl.when(i == 0)
    def _():
        wv = w_ref[...]
        m = jnp.max(wv)
        e = jnp.exp(wv - m)
        sw_ref[...] = e / jnp.sum(e)

    o_ref[...] = jnp.sum(x_ref[...] * sw_ref[...], axis=1, keepdims=True)


def kernel(x, w, k):
    del k
    t, n = x.shape
    bt = min(_BT, t)
    y = pl.pallas_call(
        _mv_body,
        grid=(t // bt,),
        in_specs=[
            pl.BlockSpec((1, n), lambda i: (0, 0)),
            pl.BlockSpec((bt, n), lambda i: (i, 0)),
        ],
        out_specs=pl.BlockSpec((bt, 1), lambda i: (i, 0)),
        out_shape=jax.ShapeDtypeStruct((t, 1), jnp.float32),
        scratch_shapes=[pltpu.VMEM((1, n), jnp.float32)],
    )(w.reshape(1, n), x)
    return y.reshape(t)
